# baseline (speedup 1.0000x reference)
import jax
import jax.numpy as jnp
from jax.experimental import pallas as pl

_DIST2_THRESH = 0.2 * 0.2
_CONF_THRESH = 0.7
_BIG = 1e30


def _adj_kernel(cent_ref, trow_ref, tcol_ref, odiag_ref, out_ref):
    cm = cent_ref[0]
    g = jax.lax.dot_general(
        cm,
        cm,
        dimension_numbers=(((1,), (1,)), ((), ())),
        preferred_element_type=jnp.float32,
        precision=jax.lax.Precision.HIGHEST,
    )
    t = tcol_ref[0] + trow_ref[0]
    out_ref[0] = jnp.where(g < t, odiag_ref[...], 0.0)


def kernel(centroid, obj_conf):
    B, N, _ = centroid.shape
    n2 = jnp.sum(centroid * centroid, axis=-1)
    conf_ok = obj_conf >= _CONF_THRESH
    half = (n2 - 0.5 * _DIST2_THRESH) * 0.5
    t_row = jnp.where(conf_ok, half, -jnp.inf)
    dist_enabled = (jnp.arange(B)[:, None] >= 2) & (jnp.arange(N)[None, :] >= 2)
    t_col = jnp.where(conf_ok, jnp.where(dist_enabled, half, _BIG), -jnp.inf)
    odiag = 1.0 - jnp.eye(N, dtype=jnp.float32)
    return pl.pallas_call(
        _adj_kernel,
        grid=(B,),
        in_specs=[
            pl.BlockSpec((1, N, 3), lambda b: (b, 0, 0)),
            pl.BlockSpec((1, 1, N), lambda b: (b, 0, 0)),
            pl.BlockSpec((1, N, 1), lambda b: (b, 0, 0)),
            pl.BlockSpec((N, N), lambda b: (0, 0)),
        ],
        out_specs=pl.BlockSpec((1, N, N), lambda b: (b, 0, 0)),
        out_shape=jax.ShapeDtypeStruct((B, N, N), jnp.float32),
    )(
        centroid,
        t_row[:, None, :],
        t_col[:, :, None],
        odiag,
    )


# speedup vs baseline: 1.1414x; 1.1414x over previous
import jax
import jax.numpy as jnp
from jax.experimental import pallas as pl

_DIST2_THRESH = 0.2 * 0.2
_CONF_THRESH = 0.7
_BIG = 1e30


def _adj_kernel(row_ref, col_ref, trow_ref, tcol_ref, odiag_ref, out_ref):
    x = row_ref[0, 0:1, :]
    y = row_ref[0, 1:2, :]
    z = row_ref[0, 2:3, :]
    xc = col_ref[0, :, 0:1]
    yc = col_ref[0, :, 1:2]
    zc = col_ref[0, :, 2:3]
    dx = xc - x
    dy = yc - y
    dz = zc - z
    d2 = dx * dx + dy * dy + dz * dz
    t = tcol_ref[0] + trow_ref[0]
    out_ref[0] = jnp.where(d2 > t, odiag_ref[...], 0.0)


def kernel(centroid, obj_conf):
    B, N, _ = centroid.shape
    conf_ok = obj_conf >= _CONF_THRESH
    half = jnp.full_like(obj_conf, 0.5 * _DIST2_THRESH)
    t_row = jnp.where(conf_ok, half, jnp.inf)
    dist_enabled = (jnp.arange(B)[:, None] >= 2) & (jnp.arange(N)[None, :] >= 2)
    t_col = jnp.where(conf_ok, jnp.where(dist_enabled, half, -_BIG), jnp.inf)
    odiag = 1.0 - jnp.eye(N, dtype=jnp.float32)
    rows = jnp.transpose(centroid, (0, 2, 1))
    return pl.pallas_call(
        _adj_kernel,
        grid=(B,),
        in_specs=[
            pl.BlockSpec((1, 3, N), lambda b: (b, 0, 0)),
            pl.BlockSpec((1, N, 3), lambda b: (b, 0, 0)),
            pl.BlockSpec((1, 1, N), lambda b: (b, 0, 0)),
            pl.BlockSpec((1, N, 1), lambda b: (b, 0, 0)),
            pl.BlockSpec((N, N), lambda b: (0, 0)),
        ],
        out_specs=pl.BlockSpec((1, N, N), lambda b: (b, 0, 0)),
        out_shape=jax.ShapeDtypeStruct((B, N, N), jnp.float32),
    )(
        rows,
        centroid,
        t_row[:, None, :],
        t_col[:, :, None],
        odiag,
    )


# speedup vs baseline: 1.1730x; 1.0277x over previous
import jax
import jax.numpy as jnp
from jax.experimental import pallas as pl
from jax.experimental.pallas import tpu as pltpu

_DIST2_THRESH = 0.2 * 0.2
_CONF_THRESH = 0.7
_BIG = 1e30


def _adj_kernel(row_ref, col_ref, trow_ref, tcol_ref, out_ref, odiag_ref):
    n = out_ref.shape[1]

    @pl.when(pl.program_id(0) == 0)
    def _init():
        rows = jax.lax.broadcasted_iota(jnp.int32, (n, n), 0)
        cols = jax.lax.broadcasted_iota(jnp.int32, (n, n), 1)
        odiag_ref[...] = (rows != cols).astype(jnp.float32)

    x = row_ref[0, 0:1, :]
    y = row_ref[0, 1:2, :]
    z = row_ref[0, 2:3, :]
    xc = col_ref[0, :, 0:1]
    yc = col_ref[0, :, 1:2]
    zc = col_ref[0, :, 2:3]
    dx = xc - x
    dy = yc - y
    dz = zc - z
    d2 = dx * dx + dy * dy + dz * dz
    t = tcol_ref[0] + trow_ref[0]
    out_ref[0] = jnp.where(d2 > t, odiag_ref[...], 0.0)


def kernel(centroid, obj_conf):
    B, N, _ = centroid.shape
    conf_ok = obj_conf >= _CONF_THRESH
    half = jnp.full_like(obj_conf, 0.5 * _DIST2_THRESH)
    t_row = jnp.where(conf_ok, half, jnp.inf)
    dist_enabled = (jnp.arange(B)[:, None] >= 2) & (jnp.arange(N)[None, :] >= 2)
    t_col = jnp.where(conf_ok, jnp.where(dist_enabled, half, -_BIG), jnp.inf)
    rows = jnp.transpose(centroid, (0, 2, 1))
    return pl.pallas_call(
        _adj_kernel,
        grid=(B,),
        in_specs=[
            pl.BlockSpec((1, 3, N), lambda b: (b, 0, 0)),
            pl.BlockSpec((1, N, 3), lambda b: (b, 0, 0)),
            pl.BlockSpec((1, 1, N), lambda b: (b, 0, 0)),
            pl.BlockSpec((1, N, 1), lambda b: (b, 0, 0)),
        ],
        out_specs=pl.BlockSpec((1, N, N), lambda b: (b, 0, 0)),
        out_shape=jax.ShapeDtypeStruct((B, N, N), jnp.float32),
        scratch_shapes=[pltpu.VMEM((N, N), jnp.float32)],
    )(
        rows,
        centroid,
        t_row[:, None, :],
        t_col[:, :, None],
    )


# speedup vs baseline: 1.5869x; 1.3528x over previous
import jax
import jax.numpy as jnp
from jax.experimental import pallas as pl
from jax.experimental.pallas import tpu as pltpu

_DIST2_THRESH = 0.2 * 0.2
_CONF_THRESH = 0.7
_BIG = 1e30


def _adj_kernel(in_ref, out_ref, odiag_ref):
    n = out_ref.shape[1]

    @pl.when(pl.program_id(0) == 0)
    def _init():
        rows = jax.lax.broadcasted_iota(jnp.int32, (n, n), 0)
        cols = jax.lax.broadcasted_iota(jnp.int32, (n, n), 1)
        odiag_ref[...] = (rows != cols).astype(jnp.float32)

    x = in_ref[0, 0:1, :]
    y = in_ref[0, 1:2, :]
    z = in_ref[0, 2:3, :]
    t_row = in_ref[0, 3:4, :]
    xc = jnp.transpose(in_ref[0, 0:1, :])
    yc = jnp.transpose(in_ref[0, 1:2, :])
    zc = jnp.transpose(in_ref[0, 2:3, :])
    tc = jnp.transpose(in_ref[0, 4:5, :])
    dx = xc - x
    dy = yc - y
    dz = zc - z
    d2 = dx * dx + dy * dy + dz * dz
    t = tc + t_row
    out_ref[0] = jnp.where(d2 > t, odiag_ref[...], 0.0)


def kernel(centroid, obj_conf):
    B, N, _ = centroid.shape
    conf_ok = obj_conf >= _CONF_THRESH
    half = jnp.full_like(obj_conf, 0.5 * _DIST2_THRESH)
    t_row = jnp.where(conf_ok, half, jnp.inf)
    dist_enabled = (jnp.arange(B)[:, None] >= 2) & (jnp.arange(N)[None, :] >= 2)
    t_col = jnp.where(conf_ok, jnp.where(dist_enabled, half, -_BIG), jnp.inf)
    packed = jnp.concatenate(
        [
            jnp.transpose(centroid, (0, 2, 1)),
            t_row[:, None, :],
            t_col[:, None, :],
        ],
        axis=1,
    )
    return pl.pallas_call(
        _adj_kernel,
        grid=(B,),
        in_specs=[pl.BlockSpec((1, 5, N), lambda b: (b, 0, 0))],
        out_specs=pl.BlockSpec((1, N, N), lambda b: (b, 0, 0)),
        out_shape=jax.ShapeDtypeStruct((B, N, N), jnp.float32),
        scratch_shapes=[pltpu.VMEM((N, N), jnp.float32)],
    )(packed)
